# baseline (speedup 1.0000x reference)
import functools

import jax
import jax.numpy as jnp
from jax import lax
from jax.experimental import pallas as pl
from jax.experimental.pallas import tpu as pltpu
from jax.experimental.pallas import tpu_sc as plsc

N = 10000
E = 320000
IN_DIM = 29
D = 128
AH = 256
CHD = 128
A = 2048
L = 6

NC = 2
NS = 16
NW = NC * NS

NR = 10240
ROWS_PER_TILE = NR // NS

EPW = 10240
EP = NW * EPW
CH_E = 128
NCHUNK = EPW // CH_E
NQUAD = NCHUNK // 4
EPA = EP + 4 * CH_E

R = 512
GRID = NR // R

@functools.lru_cache(maxsize=1)
def _sc_mesh():
    return plsc.VectorSubcoreMesh(
        core_axis_name="c", subcore_axis_name="s", num_cores=NC, num_subcores=NS
    )


def _mp_body(w, p_hbm, src_hbm, dst_hbm, out_hbm,
             idx_s0, idx_s1, idx_s2, idx_s3, idx_d0, idx_d1, idx_d2, idx_d3,
             rows0, rows1, agg,
             gsem0, gsem1, ssem0, ssem1, isem0, isem1, isem2, isem3):
    c = lax.axis_index("c")
    s = lax.axis_index("s")
    wid = s * NC + c
    base = s * ROWS_PER_TILE
    ebase = wid * EPW

    zero16 = jnp.zeros((16,), jnp.float32)

    def zrow(i, carry):
        for j in range(w // 16):
            rows0[i, pl.ds(j * 16, 16)] = zero16
        return carry

    lax.fori_loop(0, CH_E, zrow, 0)
    for k in range(ROWS_PER_TILE // CH_E):
        pltpu.sync_copy(rows0, agg.at[pl.ds(base + k * CH_E, CH_E)])

    pltpu.sync_copy(src_hbm.at[pl.ds(ebase, CH_E)], idx_s0)
    pltpu.sync_copy(dst_hbm.at[pl.ds(ebase, CH_E)], idx_d0)
    pltpu.sync_copy(src_hbm.at[pl.ds(ebase + CH_E, CH_E)], idx_s1)
    pltpu.sync_copy(dst_hbm.at[pl.ds(ebase + CH_E, CH_E)], idx_d1)
    pltpu.sync_copy(src_hbm.at[pl.ds(ebase + 2 * CH_E, CH_E)], idx_s2)
    pltpu.sync_copy(dst_hbm.at[pl.ds(ebase + 2 * CH_E, CH_E)], idx_d2)
    pltpu.sync_copy(src_hbm.at[pl.ds(ebase + 3 * CH_E, CH_E)], idx_s3)
    pltpu.sync_copy(dst_hbm.at[pl.ds(ebase + 3 * CH_E, CH_E)], idx_d3)
    pltpu.async_copy(p_hbm.at[idx_s0], rows0, gsem0)
    pltpu.async_copy(p_hbm.at[idx_s1], rows1, gsem1)

    plsc.subcore_barrier()

    def quad(k, carry):
        nxt = ebase + (4 * k + 4) * CH_E
        pltpu.make_async_copy(p_hbm.at[idx_s0], rows0, gsem0).wait()
        s0 = pltpu.async_copy(rows0, agg.at[idx_d0], ssem0, add=True)
        pltpu.make_async_copy(p_hbm.at[idx_s1], rows1, gsem1).wait()
        s1 = pltpu.async_copy(rows1, agg.at[idx_d1], ssem1, add=True)
        s0.wait()
        g2 = pltpu.async_copy(p_hbm.at[idx_s2], rows0, gsem0)
        ia0s = pltpu.async_copy(src_hbm.at[pl.ds(nxt, CH_E)], idx_s0, isem0)
        ia0d = pltpu.async_copy(dst_hbm.at[pl.ds(nxt, CH_E)], idx_d0, isem0)
        s1.wait()
        g3 = pltpu.async_copy(p_hbm.at[idx_s3], rows1, gsem1)
        ia1s = pltpu.async_copy(src_hbm.at[pl.ds(nxt + CH_E, CH_E)], idx_s1, isem1)
        ia1d = pltpu.async_copy(dst_hbm.at[pl.ds(nxt + CH_E, CH_E)], idx_d1, isem1)
        g2.wait()
        s2 = pltpu.async_copy(rows0, agg.at[idx_d2], ssem0, add=True)
        g3.wait()
        s3 = pltpu.async_copy(rows1, agg.at[idx_d3], ssem1, add=True)
        s2.wait()
        ib0s = pltpu.async_copy(src_hbm.at[pl.ds(nxt + 2 * CH_E, CH_E)], idx_s2, isem2)
        ib0d = pltpu.async_copy(dst_hbm.at[pl.ds(nxt + 2 * CH_E, CH_E)], idx_d2, isem2)
        ia0s.wait()
        ia0d.wait()
        pltpu.async_copy(p_hbm.at[idx_s0], rows0, gsem0)
        s3.wait()
        ib1s = pltpu.async_copy(src_hbm.at[pl.ds(nxt + 3 * CH_E, CH_E)], idx_s3, isem3)
        ib1d = pltpu.async_copy(dst_hbm.at[pl.ds(nxt + 3 * CH_E, CH_E)], idx_d3, isem3)
        ia1s.wait()
        ia1d.wait()
        pltpu.async_copy(p_hbm.at[idx_s1], rows1, gsem1)
        ib0s.wait()
        ib0d.wait()
        ib1s.wait()
        ib1d.wait()
        return carry

    lax.fori_loop(0, NQUAD, quad, 0)

    pltpu.make_async_copy(p_hbm.at[idx_s0], rows0, gsem0).wait()
    pltpu.make_async_copy(p_hbm.at[idx_s1], rows1, gsem1).wait()

    plsc.subcore_barrier()

    pltpu.sync_copy(
        agg.at[pl.ds(base, ROWS_PER_TILE)],
        out_hbm.at[c, pl.ds(base, ROWS_PER_TILE)],
    )


def _mp_call(p, srcp, dstp):
    w = p.shape[1]
    f = pl.kernel(
        functools.partial(_mp_body, w),
        out_type=[jax.ShapeDtypeStruct((NC, NR, w), jnp.float32)],
        mesh=_sc_mesh(),
        scratch_types=(
            [pltpu.VMEM((CH_E,), jnp.int32)] * 8
            + [
                pltpu.VMEM((CH_E, w), jnp.float32),
                pltpu.VMEM((CH_E, w), jnp.float32),
                pltpu.VMEM_SHARED((NR, w), jnp.float32),
            ]
            + [pltpu.SemaphoreType.DMA] * 8
        ),
        name="sc_mp%d" % w,
    )
    return f(p, srcp, dstp)


def _combine0_body(s_ref, parts_ref, wn0_ref, wn_ref, ws_ref, b_ref,
                   p_out, s_out, inv_out):
    a32 = parts_ref[0] + parts_ref[1]
    inv = 1.0 / jnp.clip(a32[:, 31:32], 1.0, None)
    aggn = jnp.dot(a32, wn0_ref[...], preferred_element_type=jnp.float32) * inv
    h = jnp.maximum(s_ref[...] + aggn, 0.0)
    p_out[...] = jnp.dot(h, wn_ref[...], preferred_element_type=jnp.float32)
    s_out[...] = (
        jnp.dot(h, ws_ref[...], preferred_element_type=jnp.float32) + b_ref[...]
    )
    inv_out[...] = jnp.broadcast_to(inv, (R, 8))


def _combine0(s, parts32, wn0, wn, ws, b):
    return pl.pallas_call(
        _combine0_body,
        grid=(GRID,),
        in_specs=[
            pl.BlockSpec((R, D), lambda i: (i, 0)),
            pl.BlockSpec((2, R, D), lambda i: (0, i, 0)),
            pl.BlockSpec((D, D), lambda i: (0, 0)),
            pl.BlockSpec((D, D), lambda i: (0, 0)),
            pl.BlockSpec((D, D), lambda i: (0, 0)),
            pl.BlockSpec((1, D), lambda i: (0, 0)),
        ],
        out_specs=[
            pl.BlockSpec((R, D), lambda i: (i, 0)),
            pl.BlockSpec((R, D), lambda i: (i, 0)),
            pl.BlockSpec((R, 8), lambda i: (i, 0)),
        ],
        out_shape=[
            jax.ShapeDtypeStruct((NR, D), jnp.float32),
            jax.ShapeDtypeStruct((NR, D), jnp.float32),
            jax.ShapeDtypeStruct((NR, 8), jnp.float32),
        ],
    )(s, parts32, wn0, wn, ws, b)


def _premul0_body(x_ref, ws_ref, b_ref, s_out):
    s_out[...] = (
        jnp.dot(x_ref[...], ws_ref[...], preferred_element_type=jnp.float32)
        + b_ref[...]
    )


def _premul0(xp, ws0, b0):
    return pl.pallas_call(
        _premul0_body,
        grid=(GRID,),
        in_specs=[
            pl.BlockSpec((R, 32), lambda i: (i, 0)),
            pl.BlockSpec((32, D), lambda i: (0, 0)),
            pl.BlockSpec((1, D), lambda i: (0, 0)),
        ],
        out_specs=pl.BlockSpec((R, D), lambda i: (i, 0)),
        out_shape=jax.ShapeDtypeStruct((NR, D), jnp.float32),
    )(xp, ws0, b0)


def _combine_body(s_ref, parts_ref, inv_ref, wn_ref, ws_ref, b_ref, p_out, s_out):
    aggn = (parts_ref[0] + parts_ref[1]) * inv_ref[:, 0:1]
    h = jnp.maximum(s_ref[...] + aggn, 0.0)
    p_out[...] = jnp.dot(h, wn_ref[...], preferred_element_type=jnp.float32)
    s_out[...] = (
        jnp.dot(h, ws_ref[...], preferred_element_type=jnp.float32) + b_ref[...]
    )


def _combine(s, parts, inv8, wn, ws, b):
    return pl.pallas_call(
        _combine_body,
        grid=(GRID,),
        in_specs=[
            pl.BlockSpec((R, D), lambda i: (i, 0)),
            pl.BlockSpec((2, R, D), lambda i: (0, i, 0)),
            pl.BlockSpec((R, 8), lambda i: (i, 0)),
            pl.BlockSpec((D, D), lambda i: (0, 0)),
            pl.BlockSpec((D, D), lambda i: (0, 0)),
            pl.BlockSpec((1, D), lambda i: (0, 0)),
        ],
        out_specs=[
            pl.BlockSpec((R, D), lambda i: (i, 0)),
            pl.BlockSpec((R, D), lambda i: (i, 0)),
        ],
        out_shape=[
            jax.ShapeDtypeStruct((NR, D), jnp.float32),
            jax.ShapeDtypeStruct((NR, D), jnp.float32),
        ],
    )(s, parts, inv8, wn, ws, b)


def _heads_body(s_ref, parts_ref, inv_ref, cw1_ref, cb1_ref, cw2_ref, cb2_ref,
                aw1_ref, ab1_ref, aw2_ref, ab2_ref, vs_out, xf_out):
    i = pl.program_id(0)
    aggn = (parts_ref[0] + parts_ref[1]) * inv_ref[:, 0:1]
    h = jnp.maximum(s_ref[...] + aggn, 0.0)
    hc = jnp.maximum(
        jnp.dot(h, cw1_ref[...], preferred_element_type=jnp.float32)
        + cb1_ref[...],
        0.0,
    )
    vs = jnp.dot(hc, cw2_ref[...], preferred_element_type=jnp.float32) + cb2_ref[...]
    rowid = lax.broadcasted_iota(jnp.int32, (R, 8), 0) + i * R
    vs_out[...] = jnp.where(rowid < N, vs, -1e30)
    ha = jnp.maximum(
        jnp.dot(h, aw1_ref[...], preferred_element_type=jnp.float32)
        + ab1_ref[...],
        0.0,
    )
    xf_out[...] = (
        jnp.dot(ha, aw2_ref[...], preferred_element_type=jnp.float32) + ab2_ref[...]
    )


def _heads(s, parts, inv8, cw1, cb1, cw2p, cb2r, aw1, ab1, aw2, ab2):
    return pl.pallas_call(
        _heads_body,
        grid=(GRID,),
        in_specs=[
            pl.BlockSpec((R, D), lambda i: (i, 0)),
            pl.BlockSpec((2, R, D), lambda i: (0, i, 0)),
            pl.BlockSpec((R, 8), lambda i: (i, 0)),
            pl.BlockSpec((D, CHD), lambda i: (0, 0)),
            pl.BlockSpec((1, CHD), lambda i: (0, 0)),
            pl.BlockSpec((CHD, 8), lambda i: (0, 0)),
            pl.BlockSpec((1, 8), lambda i: (0, 0)),
            pl.BlockSpec((D, AH), lambda i: (0, 0)),
            pl.BlockSpec((1, AH), lambda i: (0, 0)),
            pl.BlockSpec((AH, A), lambda i: (0, 0)),
            pl.BlockSpec((1, A), lambda i: (0, 0)),
        ],
        out_specs=[
            pl.BlockSpec((R, 8), lambda i: (i, 0)),
            pl.BlockSpec((R, A), lambda i: (i, 0)),
        ],
        out_shape=[
            jax.ShapeDtypeStruct((NR, 8), jnp.float32),
            jax.ShapeDtypeStruct((NR, A), jnp.float32),
        ],
    )(s, parts, inv8, cw1, cb1, cw2p, cb2r, aw1, ab1, aw2, ab2)


def _softmax_body(vs_ref, out_ref):
    v = vs_ref[:, 0:1]
    m = jnp.max(v)
    e = jnp.exp(vs_ref[...] - m)
    ssum = jnp.sum(e[:, 0:1])
    out_ref[...] = e / ssum


def _softmax(vs8):
    return pl.pallas_call(
        _softmax_body,
        out_shape=jax.ShapeDtypeStruct((NR, 8), jnp.float32),
    )(vs8)


def kernel(x, edge_index, params):
    gnn = params["gnn"]
    act = params["actor"]
    cri = params["critic"]

    src = edge_index[0]
    dst = edge_index[1]
    pad = EPA - E
    padi = jnp.arange(pad, dtype=jnp.int32)
    srcp = jnp.concatenate([src, padi % N])
    dstp = jnp.concatenate([dst, N + padi % (NR - N)])

    xp = jnp.pad(x, ((0, NR - N), (0, 32 - IN_DIM)))
    x128 = jnp.pad(xp.at[:, 31].set(1.0), ((0, 0), (0, D - 32)))
    wn0 = jnp.pad(gnn[0]["W_neigh"], ((0, D - IN_DIM), (0, 0)))
    ws0 = jnp.pad(gnn[0]["W_self"], ((0, 32 - IN_DIM), (0, 0)))
    b0 = gnn[0]["b"].reshape(1, D)

    s = _premul0(xp, ws0, b0)
    (parts32,) = _mp_call(x128, srcp, dstp)
    p, s, inv8 = _combine0(s, parts32, wn0, gnn[1]["W_neigh"],
                           gnn[1]["W_self"], gnn[1]["b"].reshape(1, D))
    (parts,) = _mp_call(p, srcp, dstp)

    for l in range(2, L):
        wn = gnn[l]["W_neigh"]
        ws = gnn[l]["W_self"]
        b = gnn[l]["b"].reshape(1, D)
        p, s = _combine(s, parts, inv8, wn, ws, b)
        (parts,) = _mp_call(p, srcp, dstp)

    cw1 = cri["W1"]
    cb1 = cri["b1"].reshape(1, CHD)
    cw2p = jnp.pad(cri["W2"], ((0, 0), (0, 7)))
    cb2r = jnp.broadcast_to(cri["b2"].reshape(1, 1), (1, 8))
    aw1 = act["W1"]
    ab1 = act["b1"].reshape(1, AH)
    aw2 = act["W2"]
    ab2 = act["b2"].reshape(1, A)

    vs8, xf = _heads(s, parts, inv8, cw1, cb1, cw2p, cb2r, aw1, ab1, aw2, ab2)
    prob8 = _softmax(vs8)

    node_vs = vs8[:N, 0]
    node_prob = prob8[:N, 0]
    xfer_logits = xf[:N]
    return (node_vs, node_prob, xfer_logits)


# speedup vs baseline: 1.1052x; 1.1052x over previous
import functools

import jax
import jax.numpy as jnp
from jax import lax
from jax.experimental import pallas as pl
from jax.experimental.pallas import tpu as pltpu
from jax.experimental.pallas import tpu_sc as plsc

N = 10000
E = 320000
IN_DIM = 29
D = 128
AH = 256
CHD = 128
A = 2048
L = 6

NC = 2
NS = 16
NW = NC * NS

NR = 10240
ROWS_PER_TILE = NR // NS

EPW = 10240
EP = NW * EPW
CH_E = 128
NCHUNK = EPW // CH_E
NPAIR = NCHUNK // 2
EPA = EP + 2 * CH_E

R = 512
GRID = NR // R

@functools.lru_cache(maxsize=1)
def _sc_mesh():
    return plsc.VectorSubcoreMesh(
        core_axis_name="c", subcore_axis_name="s", num_cores=NC, num_subcores=NS
    )


def _mp_body(w, p_hbm, src_hbm, dst_hbm, out_hbm,
             idx_s0, idx_s1, idx_d0, idx_d1, rows0, rows1, agg,
             gsem0, gsem1, ssem0, ssem1, isem0, isem1):
    c = lax.axis_index("c")
    s = lax.axis_index("s")
    wid = s * NC + c
    base = s * ROWS_PER_TILE
    ebase = wid * EPW

    zero16 = jnp.zeros((16,), jnp.float32)

    def zrow(i, carry):
        for j in range(w // 16):
            rows0[i, pl.ds(j * 16, 16)] = zero16
        return carry

    lax.fori_loop(0, CH_E, zrow, 0)
    for k in range(ROWS_PER_TILE // CH_E):
        pltpu.sync_copy(rows0, agg.at[pl.ds(base + k * CH_E, CH_E)])

    pltpu.sync_copy(src_hbm.at[pl.ds(ebase, CH_E)], idx_s0)
    pltpu.sync_copy(dst_hbm.at[pl.ds(ebase, CH_E)], idx_d0)
    pltpu.sync_copy(src_hbm.at[pl.ds(ebase + CH_E, CH_E)], idx_s1)
    pltpu.sync_copy(dst_hbm.at[pl.ds(ebase + CH_E, CH_E)], idx_d1)
    pltpu.async_copy(p_hbm.at[idx_s0], rows0, gsem0)
    pltpu.async_copy(p_hbm.at[idx_s1], rows1, gsem1)

    plsc.subcore_barrier()

    def pair(k, carry):
        nxt = ebase + (2 * k + 2) * CH_E
        pltpu.make_async_copy(p_hbm.at[idx_s0], rows0, gsem0).wait()
        s0 = pltpu.async_copy(rows0, agg.at[idx_d0], ssem0, add=True)
        pltpu.make_async_copy(p_hbm.at[idx_s1], rows1, gsem1).wait()
        s0.wait()
        i0s = pltpu.async_copy(src_hbm.at[pl.ds(nxt, CH_E)], idx_s0, isem0)
        i0d = pltpu.async_copy(dst_hbm.at[pl.ds(nxt, CH_E)], idx_d0, isem1)
        s1 = pltpu.async_copy(rows1, agg.at[idx_d1], ssem1, add=True)
        i0s.wait()
        i0d.wait()
        pltpu.async_copy(p_hbm.at[idx_s0], rows0, gsem0)
        s1.wait()
        i1s = pltpu.async_copy(src_hbm.at[pl.ds(nxt + CH_E, CH_E)], idx_s1, isem0)
        i1d = pltpu.async_copy(dst_hbm.at[pl.ds(nxt + CH_E, CH_E)], idx_d1, isem1)
        i1s.wait()
        i1d.wait()
        pltpu.async_copy(p_hbm.at[idx_s1], rows1, gsem1)
        return carry

    lax.fori_loop(0, NPAIR, pair, 0)

    pltpu.make_async_copy(p_hbm.at[idx_s0], rows0, gsem0).wait()
    pltpu.make_async_copy(p_hbm.at[idx_s1], rows1, gsem1).wait()

    plsc.subcore_barrier()

    pltpu.sync_copy(
        agg.at[pl.ds(base, ROWS_PER_TILE)],
        out_hbm.at[c, pl.ds(base, ROWS_PER_TILE)],
    )


def _mp_call(p, srcp, dstp):
    w = p.shape[1]
    f = pl.kernel(
        functools.partial(_mp_body, w),
        out_type=[jax.ShapeDtypeStruct((NC, NR, w), jnp.float32)],
        mesh=_sc_mesh(),
        scratch_types=(
            [pltpu.VMEM((CH_E,), jnp.int32)] * 4
            + [
                pltpu.VMEM((CH_E, w), jnp.float32),
                pltpu.VMEM((CH_E, w), jnp.float32),
                pltpu.VMEM_SHARED((NR, w), jnp.float32),
            ]
            + [pltpu.SemaphoreType.DMA] * 6
        ),
        name="sc_mp%d" % w,
    )
    return f(p, srcp, dstp)


def _combine0_body(s_ref, parts_ref, wn0_ref, wn_ref, ws_ref, b_ref,
                   p_out, s_out, inv_out):
    a32 = parts_ref[0] + parts_ref[1]
    inv = 1.0 / jnp.clip(a32[:, 31:32], 1.0, None)
    aggn = jnp.dot(a32, wn0_ref[...], preferred_element_type=jnp.float32) * inv
    h = jnp.maximum(s_ref[...] + aggn, 0.0)
    p_out[...] = jnp.dot(h, wn_ref[...], preferred_element_type=jnp.float32)
    s_out[...] = (
        jnp.dot(h, ws_ref[...], preferred_element_type=jnp.float32) + b_ref[...]
    )
    inv_out[...] = jnp.broadcast_to(inv, (R, 8))


def _combine0(s, parts32, wn0, wn, ws, b):
    return pl.pallas_call(
        _combine0_body,
        grid=(GRID,),
        in_specs=[
            pl.BlockSpec((R, D), lambda i: (i, 0)),
            pl.BlockSpec((2, R, D), lambda i: (0, i, 0)),
            pl.BlockSpec((D, D), lambda i: (0, 0)),
            pl.BlockSpec((D, D), lambda i: (0, 0)),
            pl.BlockSpec((D, D), lambda i: (0, 0)),
            pl.BlockSpec((1, D), lambda i: (0, 0)),
        ],
        out_specs=[
            pl.BlockSpec((R, D), lambda i: (i, 0)),
            pl.BlockSpec((R, D), lambda i: (i, 0)),
            pl.BlockSpec((R, 8), lambda i: (i, 0)),
        ],
        out_shape=[
            jax.ShapeDtypeStruct((NR, D), jnp.float32),
            jax.ShapeDtypeStruct((NR, D), jnp.float32),
            jax.ShapeDtypeStruct((NR, 8), jnp.float32),
        ],
    )(s, parts32, wn0, wn, ws, b)


def _premul0_body(x_ref, ws_ref, b_ref, s_out):
    s_out[...] = (
        jnp.dot(x_ref[...], ws_ref[...], preferred_element_type=jnp.float32)
        + b_ref[...]
    )


def _premul0(xp, ws0, b0):
    return pl.pallas_call(
        _premul0_body,
        grid=(GRID,),
        in_specs=[
            pl.BlockSpec((R, 32), lambda i: (i, 0)),
            pl.BlockSpec((32, D), lambda i: (0, 0)),
            pl.BlockSpec((1, D), lambda i: (0, 0)),
        ],
        out_specs=pl.BlockSpec((R, D), lambda i: (i, 0)),
        out_shape=jax.ShapeDtypeStruct((NR, D), jnp.float32),
    )(xp, ws0, b0)


def _combine_body(s_ref, parts_ref, inv_ref, wn_ref, ws_ref, b_ref, p_out, s_out):
    aggn = (parts_ref[0] + parts_ref[1]) * inv_ref[:, 0:1]
    h = jnp.maximum(s_ref[...] + aggn, 0.0)
    p_out[...] = jnp.dot(h, wn_ref[...], preferred_element_type=jnp.float32)
    s_out[...] = (
        jnp.dot(h, ws_ref[...], preferred_element_type=jnp.float32) + b_ref[...]
    )


def _combine(s, parts, inv8, wn, ws, b):
    return pl.pallas_call(
        _combine_body,
        grid=(GRID,),
        in_specs=[
            pl.BlockSpec((R, D), lambda i: (i, 0)),
            pl.BlockSpec((2, R, D), lambda i: (0, i, 0)),
            pl.BlockSpec((R, 8), lambda i: (i, 0)),
            pl.BlockSpec((D, D), lambda i: (0, 0)),
            pl.BlockSpec((D, D), lambda i: (0, 0)),
            pl.BlockSpec((1, D), lambda i: (0, 0)),
        ],
        out_specs=[
            pl.BlockSpec((R, D), lambda i: (i, 0)),
            pl.BlockSpec((R, D), lambda i: (i, 0)),
        ],
        out_shape=[
            jax.ShapeDtypeStruct((NR, D), jnp.float32),
            jax.ShapeDtypeStruct((NR, D), jnp.float32),
        ],
    )(s, parts, inv8, wn, ws, b)


def _heads_body(s_ref, parts_ref, inv_ref, cw1_ref, cb1_ref, cw2_ref, cb2_ref,
                aw1_ref, ab1_ref, aw2_ref, ab2_ref, vs_out, xf_out):
    i = pl.program_id(0)
    aggn = (parts_ref[0] + parts_ref[1]) * inv_ref[:, 0:1]
    h = jnp.maximum(s_ref[...] + aggn, 0.0)
    hc = jnp.maximum(
        jnp.dot(h, cw1_ref[...], preferred_element_type=jnp.float32)
        + cb1_ref[...],
        0.0,
    )
    vs = jnp.dot(hc, cw2_ref[...], preferred_element_type=jnp.float32) + cb2_ref[...]
    rowid = lax.broadcasted_iota(jnp.int32, (R, 8), 0) + i * R
    vs_out[...] = jnp.where(rowid < N, vs, -1e30)
    ha = jnp.maximum(
        jnp.dot(h, aw1_ref[...], preferred_element_type=jnp.float32)
        + ab1_ref[...],
        0.0,
    )
    xf_out[...] = (
        jnp.dot(ha, aw2_ref[...], preferred_element_type=jnp.float32) + ab2_ref[...]
    )


def _heads(s, parts, inv8, cw1, cb1, cw2p, cb2r, aw1, ab1, aw2, ab2):
    return pl.pallas_call(
        _heads_body,
        grid=(GRID,),
        in_specs=[
            pl.BlockSpec((R, D), lambda i: (i, 0)),
            pl.BlockSpec((2, R, D), lambda i: (0, i, 0)),
            pl.BlockSpec((R, 8), lambda i: (i, 0)),
            pl.BlockSpec((D, CHD), lambda i: (0, 0)),
            pl.BlockSpec((1, CHD), lambda i: (0, 0)),
            pl.BlockSpec((CHD, 8), lambda i: (0, 0)),
            pl.BlockSpec((1, 8), lambda i: (0, 0)),
            pl.BlockSpec((D, AH), lambda i: (0, 0)),
            pl.BlockSpec((1, AH), lambda i: (0, 0)),
            pl.BlockSpec((AH, A), lambda i: (0, 0)),
            pl.BlockSpec((1, A), lambda i: (0, 0)),
        ],
        out_specs=[
            pl.BlockSpec((R, 8), lambda i: (i, 0)),
            pl.BlockSpec((R, A), lambda i: (i, 0)),
        ],
        out_shape=[
            jax.ShapeDtypeStruct((NR, 8), jnp.float32),
            jax.ShapeDtypeStruct((NR, A), jnp.float32),
        ],
    )(s, parts, inv8, cw1, cb1, cw2p, cb2r, aw1, ab1, aw2, ab2)


def _softmax_body(vs_ref, out_ref):
    v = vs_ref[:, 0:1]
    m = jnp.max(v)
    e = jnp.exp(vs_ref[...] - m)
    ssum = jnp.sum(e[:, 0:1])
    out_ref[...] = e / ssum


def _softmax(vs8):
    return pl.pallas_call(
        _softmax_body,
        out_shape=jax.ShapeDtypeStruct((NR, 8), jnp.float32),
    )(vs8)


def kernel(x, edge_index, params):
    gnn = params["gnn"]
    act = params["actor"]
    cri = params["critic"]

    src = edge_index[0]
    dst = edge_index[1]
    pad = EPA - E
    padi = jnp.arange(pad, dtype=jnp.int32)
    srcp = jnp.concatenate([src, padi % N])
    dstp = jnp.concatenate([dst, N + padi % (NR - N)])

    xp = jnp.pad(x, ((0, NR - N), (0, 32 - IN_DIM)))
    x128 = jnp.pad(xp.at[:, 31].set(1.0), ((0, 0), (0, D - 32)))
    wn0 = jnp.pad(gnn[0]["W_neigh"], ((0, D - IN_DIM), (0, 0)))
    ws0 = jnp.pad(gnn[0]["W_self"], ((0, 32 - IN_DIM), (0, 0)))
    b0 = gnn[0]["b"].reshape(1, D)

    s = _premul0(xp, ws0, b0)
    (parts32,) = _mp_call(x128, srcp, dstp)
    p, s, inv8 = _combine0(s, parts32, wn0, gnn[1]["W_neigh"],
                           gnn[1]["W_self"], gnn[1]["b"].reshape(1, D))
    (parts,) = _mp_call(p, srcp, dstp)

    for l in range(2, L):
        wn = gnn[l]["W_neigh"]
        ws = gnn[l]["W_self"]
        b = gnn[l]["b"].reshape(1, D)
        p, s = _combine(s, parts, inv8, wn, ws, b)
        (parts,) = _mp_call(p, srcp, dstp)

    cw1 = cri["W1"]
    cb1 = cri["b1"].reshape(1, CHD)
    cw2p = jnp.pad(cri["W2"], ((0, 0), (0, 7)))
    cb2r = jnp.broadcast_to(cri["b2"].reshape(1, 1), (1, 8))
    aw1 = act["W1"]
    ab1 = act["b1"].reshape(1, AH)
    aw2 = act["W2"]
    ab2 = act["b2"].reshape(1, A)

    vs8, xf = _heads(s, parts, inv8, cw1, cb1, cw2p, cb2r, aw1, ab1, aw2, ab2)
    prob8 = _softmax(vs8)

    node_vs = vs8[:N, 0]
    node_prob = prob8[:N, 0]
    xfer_logits = xf[:N]
    return (node_vs, node_prob, xfer_logits)


# speedup vs baseline: 1.1783x; 1.0661x over previous
import functools

import jax
import jax.numpy as jnp
from jax import lax
from jax.experimental import pallas as pl
from jax.experimental.pallas import tpu as pltpu
from jax.experimental.pallas import tpu_sc as plsc

N = 10000
E = 320000
IN_DIM = 29
D = 128
AH = 256
CHD = 128
A = 2048
L = 6

NC = 2
NS = 16
NW = NC * NS

NR = 10240
ROWS_PER_TILE = NR // NS

EPW = 10240
EP = NW * EPW
CH_E = 128
NCHUNK = EPW // CH_E
NPAIR = NCHUNK // 2
EPA = EP + 2 * CH_E
TOTCH = EPA // CH_E

R = 512
GRID = NR // R

@functools.lru_cache(maxsize=1)
def _sc_mesh():
    return plsc.VectorSubcoreMesh(
        core_axis_name="c", subcore_axis_name="s", num_cores=NC, num_subcores=NS
    )


def _mp_body(w, p_hbm, ep_hbm, out_hbm,
             idx0, idx1, rows0, rows1, agg,
             gsem0, gsem1, ssem0, ssem1, isem0, isem1):
    c = lax.axis_index("c")
    s = lax.axis_index("s")
    wid = s * NC + c
    base = s * ROWS_PER_TILE
    cbase = wid * NCHUNK

    zero16 = jnp.zeros((16,), jnp.float32)

    def zrow(i, carry):
        for j in range(w // 16):
            rows0[i, pl.ds(j * 16, 16)] = zero16
        return carry

    lax.fori_loop(0, CH_E, zrow, 0)
    for k in range(ROWS_PER_TILE // CH_E):
        pltpu.sync_copy(rows0, agg.at[pl.ds(base + k * CH_E, CH_E)])

    pltpu.sync_copy(ep_hbm.at[cbase], idx0)
    pltpu.sync_copy(ep_hbm.at[cbase + 1], idx1)
    pltpu.async_copy(p_hbm.at[idx0.at[0]], rows0, gsem0)
    pltpu.async_copy(p_hbm.at[idx1.at[0]], rows1, gsem1)

    plsc.subcore_barrier()

    def pair(k, carry):
        nxt = cbase + 2 * k + 2
        pltpu.make_async_copy(p_hbm.at[idx0.at[0]], rows0, gsem0).wait()
        s0 = pltpu.async_copy(rows0, agg.at[idx0.at[1]], ssem0, add=True)
        pltpu.make_async_copy(p_hbm.at[idx1.at[0]], rows1, gsem1).wait()
        s0.wait()
        i0 = pltpu.async_copy(ep_hbm.at[nxt], idx0, isem0)
        s1 = pltpu.async_copy(rows1, agg.at[idx1.at[1]], ssem1, add=True)
        i0.wait()
        pltpu.async_copy(p_hbm.at[idx0.at[0]], rows0, gsem0)
        s1.wait()
        i1 = pltpu.async_copy(ep_hbm.at[nxt + 1], idx1, isem1)
        i1.wait()
        pltpu.async_copy(p_hbm.at[idx1.at[0]], rows1, gsem1)
        return carry

    lax.fori_loop(0, NPAIR, pair, 0)

    pltpu.make_async_copy(p_hbm.at[idx0.at[0]], rows0, gsem0).wait()
    pltpu.make_async_copy(p_hbm.at[idx1.at[0]], rows1, gsem1).wait()

    plsc.subcore_barrier()

    pltpu.sync_copy(
        agg.at[pl.ds(base, ROWS_PER_TILE)],
        out_hbm.at[c, pl.ds(base, ROWS_PER_TILE)],
    )


def _mp_call(p, ep):
    w = p.shape[1]
    f = pl.kernel(
        functools.partial(_mp_body, w),
        out_type=[jax.ShapeDtypeStruct((NC, NR, w), jnp.float32)],
        mesh=_sc_mesh(),
        scratch_types=(
            [pltpu.VMEM((2, CH_E), jnp.int32)] * 2
            + [
                pltpu.VMEM((CH_E, w), jnp.float32),
                pltpu.VMEM((CH_E, w), jnp.float32),
                pltpu.VMEM_SHARED((NR, w), jnp.float32),
            ]
            + [pltpu.SemaphoreType.DMA] * 6
        ),
        name="sc_mp%d" % w,
    )
    return f(p, ep)


def _combine0_body(x_ref, parts_ref, ws0_ref, b0_ref, wn0_ref, wn_ref,
                   ws_ref, b_ref, p_out, s_out, inv_out):
    s0 = (
        jnp.dot(x_ref[...], ws0_ref[...], preferred_element_type=jnp.float32)
        + b0_ref[...]
    )
    a32 = parts_ref[0] + parts_ref[1]
    inv = 1.0 / jnp.clip(a32[:, 31:32], 1.0, None)
    aggn = jnp.dot(a32, wn0_ref[...], preferred_element_type=jnp.float32) * inv
    h = jnp.maximum(s0 + aggn, 0.0)
    p_out[...] = jnp.dot(h, wn_ref[...], preferred_element_type=jnp.float32)
    s_out[...] = (
        jnp.dot(h, ws_ref[...], preferred_element_type=jnp.float32) + b_ref[...]
    )
    inv_out[...] = jnp.broadcast_to(inv, (R, 8))


def _combine0(xp, parts32, ws0, b0, wn0, wn, ws, b):
    return pl.pallas_call(
        _combine0_body,
        grid=(GRID,),
        in_specs=[
            pl.BlockSpec((R, 32), lambda i: (i, 0)),
            pl.BlockSpec((2, R, D), lambda i: (0, i, 0)),
            pl.BlockSpec((32, D), lambda i: (0, 0)),
            pl.BlockSpec((1, D), lambda i: (0, 0)),
            pl.BlockSpec((D, D), lambda i: (0, 0)),
            pl.BlockSpec((D, D), lambda i: (0, 0)),
            pl.BlockSpec((D, D), lambda i: (0, 0)),
            pl.BlockSpec((1, D), lambda i: (0, 0)),
        ],
        out_specs=[
            pl.BlockSpec((R, D), lambda i: (i, 0)),
            pl.BlockSpec((R, D), lambda i: (i, 0)),
            pl.BlockSpec((R, 8), lambda i: (i, 0)),
        ],
        out_shape=[
            jax.ShapeDtypeStruct((NR, D), jnp.float32),
            jax.ShapeDtypeStruct((NR, D), jnp.float32),
            jax.ShapeDtypeStruct((NR, 8), jnp.float32),
        ],
    )(xp, parts32, ws0, b0, wn0, wn, ws, b)


def _combine_body(s_ref, parts_ref, inv_ref, wn_ref, ws_ref, b_ref, p_out, s_out):
    aggn = (parts_ref[0] + parts_ref[1]) * inv_ref[:, 0:1]
    h = jnp.maximum(s_ref[...] + aggn, 0.0)
    p_out[...] = jnp.dot(h, wn_ref[...], preferred_element_type=jnp.float32)
    s_out[...] = (
        jnp.dot(h, ws_ref[...], preferred_element_type=jnp.float32) + b_ref[...]
    )


def _combine(s, parts, inv8, wn, ws, b):
    return pl.pallas_call(
        _combine_body,
        grid=(GRID,),
        in_specs=[
            pl.BlockSpec((R, D), lambda i: (i, 0)),
            pl.BlockSpec((2, R, D), lambda i: (0, i, 0)),
            pl.BlockSpec((R, 8), lambda i: (i, 0)),
            pl.BlockSpec((D, D), lambda i: (0, 0)),
            pl.BlockSpec((D, D), lambda i: (0, 0)),
            pl.BlockSpec((1, D), lambda i: (0, 0)),
        ],
        out_specs=[
            pl.BlockSpec((R, D), lambda i: (i, 0)),
            pl.BlockSpec((R, D), lambda i: (i, 0)),
        ],
        out_shape=[
            jax.ShapeDtypeStruct((NR, D), jnp.float32),
            jax.ShapeDtypeStruct((NR, D), jnp.float32),
        ],
    )(s, parts, inv8, wn, ws, b)


def _heads_body(s_ref, parts_ref, inv_ref, cw1_ref, cb1_ref, cw2_ref, cb2_ref,
                aw1_ref, ab1_ref, aw2_ref, ab2_ref, vs_out, xf_out):
    i = pl.program_id(0)
    aggn = (parts_ref[0] + parts_ref[1]) * inv_ref[:, 0:1]
    h = jnp.maximum(s_ref[...] + aggn, 0.0)
    hc = jnp.maximum(
        jnp.dot(h, cw1_ref[...], preferred_element_type=jnp.float32)
        + cb1_ref[...],
        0.0,
    )
    vs = jnp.dot(hc, cw2_ref[...], preferred_element_type=jnp.float32) + cb2_ref[...]
    rowid = lax.broadcasted_iota(jnp.int32, (R, 8), 0) + i * R
    vs_out[...] = jnp.where(rowid < N, vs, -1e30)
    ha = jnp.maximum(
        jnp.dot(h, aw1_ref[...], preferred_element_type=jnp.float32)
        + ab1_ref[...],
        0.0,
    )
    xf_out[...] = (
        jnp.dot(ha, aw2_ref[...], preferred_element_type=jnp.float32) + ab2_ref[...]
    )


def _heads(s, parts, inv8, cw1, cb1, cw2p, cb2r, aw1, ab1, aw2, ab2):
    return pl.pallas_call(
        _heads_body,
        grid=(GRID,),
        in_specs=[
            pl.BlockSpec((R, D), lambda i: (i, 0)),
            pl.BlockSpec((2, R, D), lambda i: (0, i, 0)),
            pl.BlockSpec((R, 8), lambda i: (i, 0)),
            pl.BlockSpec((D, CHD), lambda i: (0, 0)),
            pl.BlockSpec((1, CHD), lambda i: (0, 0)),
            pl.BlockSpec((CHD, 8), lambda i: (0, 0)),
            pl.BlockSpec((1, 8), lambda i: (0, 0)),
            pl.BlockSpec((D, AH), lambda i: (0, 0)),
            pl.BlockSpec((1, AH), lambda i: (0, 0)),
            pl.BlockSpec((AH, A), lambda i: (0, 0)),
            pl.BlockSpec((1, A), lambda i: (0, 0)),
        ],
        out_specs=[
            pl.BlockSpec((R, 8), lambda i: (i, 0)),
            pl.BlockSpec((R, A), lambda i: (i, 0)),
        ],
        out_shape=[
            jax.ShapeDtypeStruct((NR, 8), jnp.float32),
            jax.ShapeDtypeStruct((N, A), jnp.float32),
        ],
    )(s, parts, inv8, cw1, cb1, cw2p, cb2r, aw1, ab1, aw2, ab2)


def _softmax_body(vs_ref, out_ref):
    v = vs_ref[:, 0:1]
    m = jnp.max(v)
    e = jnp.exp(vs_ref[...] - m)
    ssum = jnp.sum(e[:, 0:1])
    out_ref[...] = e / ssum


def _softmax(vs8):
    return pl.pallas_call(
        _softmax_body,
        out_shape=jax.ShapeDtypeStruct((NR, 8), jnp.float32),
    )(vs8)


def kernel(x, edge_index, params):
    gnn = params["gnn"]
    act = params["actor"]
    cri = params["critic"]

    src = edge_index[0]
    dst = edge_index[1]
    pad = EPA - E
    padi = jnp.arange(pad, dtype=jnp.int32)
    srcp = jnp.concatenate([src, padi % N])
    dstp = jnp.concatenate([dst, N + padi % (NR - N)])
    ep = jnp.stack(
        [srcp.reshape(TOTCH, CH_E), dstp.reshape(TOTCH, CH_E)], axis=1
    )

    xp = jnp.pad(x, ((0, NR - N), (0, 32 - IN_DIM)))
    x128 = jnp.pad(xp.at[:, 31].set(1.0), ((0, 0), (0, D - 32)))
    wn0 = jnp.pad(gnn[0]["W_neigh"], ((0, D - IN_DIM), (0, 0)))
    ws0 = jnp.pad(gnn[0]["W_self"], ((0, 32 - IN_DIM), (0, 0)))
    b0 = gnn[0]["b"].reshape(1, D)

    (parts32,) = _mp_call(x128, ep)
    p, s, inv8 = _combine0(xp, parts32, ws0, b0, wn0, gnn[1]["W_neigh"],
                           gnn[1]["W_self"], gnn[1]["b"].reshape(1, D))
    (parts,) = _mp_call(p, ep)

    for l in range(2, L):
        wn = gnn[l]["W_neigh"]
        ws = gnn[l]["W_self"]
        b = gnn[l]["b"].reshape(1, D)
        p, s = _combine(s, parts, inv8, wn, ws, b)
        (parts,) = _mp_call(p, ep)

    cw1 = cri["W1"]
    cb1 = cri["b1"].reshape(1, CHD)
    cw2p = jnp.pad(cri["W2"], ((0, 0), (0, 7)))
    cb2r = jnp.broadcast_to(cri["b2"].reshape(1, 1), (1, 8))
    aw1 = act["W1"]
    ab1 = act["b1"].reshape(1, AH)
    aw2 = act["W2"]
    ab2 = act["b2"].reshape(1, A)

    vs8, xf = _heads(s, parts, inv8, cw1, cb1, cw2p, cb2r, aw1, ab1, aw2, ab2)
    prob8 = _softmax(vs8)

    node_vs = vs8[:N, 0]
    node_prob = prob8[:N, 0]
    return (node_vs, node_prob, xf)
